# Initial kernel scaffold; baseline (speedup 1.0000x reference)
import jax
import jax.numpy as jnp
from jax.experimental import pallas as pl


def kernel(features, edge_index, W1, b1, W2, b2):
    raise NotImplementedError("write your pallas kernel here")


# speedup vs baseline: 10.2606x; 10.2606x over previous
import functools

import jax
import jax.numpy as jnp
from jax import lax
from jax.experimental import pallas as pl
from jax.experimental.pallas import tpu as pltpu
from jax.experimental.pallas import tpu_sc as plsc

N = 10000
E = 320000
D_IN = 128
HID = 16
D_OUT = 128

NC = 2
NS = 16
NW = NC * NS
EPT = E // NW
B = 80
K = EPT // B
PAD_N = 10240
RPT = PAD_N // NS

_mesh = plsc.VectorSubcoreMesh(
    core_axis_name="c", subcore_axis_name="s", num_cores=NC, num_subcores=NS
)
_sc_params = pltpu.CompilerParams(use_tc_tiling_on_sc=False)


@functools.partial(
    pl.kernel,
    out_type=jax.ShapeDtypeStruct((NC, 2, PAD_N), jnp.float32),
    mesh=_mesh,
    compiler_params=_sc_params,
    scratch_types=[
        pltpu.VMEM((K, B), jnp.int32),
        pltpu.VMEM((K, B), jnp.int32),
        pltpu.VMEM((B,), jnp.float32),
        pltpu.VMEM((RPT,), jnp.float32),
        pltpu.VMEM_SHARED((PAD_N,), jnp.float32),
        pltpu.VMEM_SHARED((PAD_N,), jnp.float32),
    ],
)
def _sc_degrees(src_hbm, dst_hbm, out_hbm, sidx, didx, ones_v, zbuf,
                dout_sh, din_sh):
    cid = lax.axis_index("c")
    sid = lax.axis_index("s")
    wid = cid * NS + sid

    zero16 = jnp.zeros((16,), jnp.float32)
    one16 = jnp.ones((16,), jnp.float32)

    def fill_z(i, _):
        zbuf[pl.ds(i * 16, 16)] = zero16
        return 0
    lax.fori_loop(0, RPT // 16, fill_z, 0)
    for i in range(B // 16):
        ones_v[pl.ds(i * 16, 16)] = one16

    pltpu.sync_copy(zbuf, dout_sh.at[pl.ds(sid * RPT, RPT)])
    pltpu.sync_copy(zbuf, din_sh.at[pl.ds(sid * RPT, RPT)])
    plsc.subcore_barrier()

    pltpu.sync_copy(src_hbm.at[wid], sidx)
    pltpu.sync_copy(dst_hbm.at[wid], didx)

    def step(j, _):
        pltpu.sync_copy(ones_v, dout_sh.at[sidx.at[j]], add=True)
        pltpu.sync_copy(ones_v, din_sh.at[didx.at[j]], add=True)
        return 0
    lax.fori_loop(0, K, step, 0)
    plsc.subcore_barrier()

    pltpu.sync_copy(dout_sh.at[pl.ds(sid * RPT, RPT)], zbuf)
    pltpu.sync_copy(zbuf, out_hbm.at[cid, 0, pl.ds(sid * RPT, RPT)])
    pltpu.sync_copy(din_sh.at[pl.ds(sid * RPT, RPT)], zbuf)
    pltpu.sync_copy(zbuf, out_hbm.at[cid, 1, pl.ds(sid * RPT, RPT)])


@functools.partial(
    pl.kernel,
    out_type=jax.ShapeDtypeStruct((NC, PAD_N, HID), jnp.float32),
    mesh=_mesh,
    compiler_params=_sc_params,
    scratch_types=[
        pltpu.VMEM((K, B), jnp.int32),
        pltpu.VMEM((K, B), jnp.int32),
        pltpu.VMEM((B, HID), jnp.float32),
        pltpu.VMEM((RPT, HID), jnp.float32),
        pltpu.VMEM_SHARED((PAD_N, HID), jnp.float32),
        pltpu.SemaphoreType.DMA,
    ],
)
def _sc_aggregate(table_hbm, src_hbm, dst_hbm, out_hbm, sidx, didx, msg,
                  zbuf, agg_sh, sem):
    cid = lax.axis_index("c")
    sid = lax.axis_index("s")
    wid = cid * NS + sid

    zero16 = jnp.zeros((16,), jnp.float32)

    def fill_z(i, _):
        zbuf[i, :] = zero16
        return 0
    lax.fori_loop(0, RPT, fill_z, 0)

    pltpu.sync_copy(zbuf, agg_sh.at[pl.ds(sid * RPT, RPT)])
    plsc.subcore_barrier()

    pltpu.sync_copy(src_hbm.at[wid], sidx)
    pltpu.sync_copy(dst_hbm.at[wid], didx)

    def step(j, _):
        pltpu.async_copy(table_hbm.at[sidx.at[j]], msg, sem).wait()
        pltpu.sync_copy(msg, agg_sh.at[didx.at[j]], add=True)
        return 0
    lax.fori_loop(0, K, step, 0)
    plsc.subcore_barrier()

    pltpu.sync_copy(agg_sh.at[pl.ds(sid * RPT, RPT)], zbuf)
    pltpu.sync_copy(zbuf, out_hbm.at[cid, pl.ds(sid * RPT, RPT)])


def _tc_scales_body(deg_ref, o_ref):
    p = deg_ref[...]
    d = p[0] + p[1]
    o_ref[...] = lax.rsqrt(jnp.maximum(d, 1.0))


_tc_scales = pl.pallas_call(
    _tc_scales_body,
    out_shape=jax.ShapeDtypeStruct((2, PAD_N), jnp.float32),
)


def _tc_layer1_body(x_ref, so_ref, w_ref, o_ref):
    x = x_ref[...] * so_ref[...]
    o_ref[...] = jnp.dot(x, w_ref[...], preferred_element_type=jnp.float32)


_tc_layer1 = pl.pallas_call(
    _tc_layer1_body,
    out_shape=jax.ShapeDtypeStruct((N, HID), jnp.float32),
)


def _tc_mid_body(a_ref, b_ref, si_ref, so_ref, bias_ref, o_ref):
    h = (a_ref[...] + b_ref[...]) * si_ref[...] + bias_ref[...]
    o_ref[...] = jnp.maximum(h, 0.0) * so_ref[...]


_tc_mid = pl.pallas_call(
    _tc_mid_body,
    out_shape=jax.ShapeDtypeStruct((N, HID), jnp.float32),
)


def _tc_layer2_body(a_ref, b_ref, w_ref, si_ref, bias_ref, o_ref):
    agg = a_ref[...] + b_ref[...]
    y = jnp.dot(agg, w_ref[...], preferred_element_type=jnp.float32)
    o_ref[...] = y * si_ref[...] + bias_ref[...]


_tc_layer2 = pl.pallas_call(
    _tc_layer2_body,
    out_shape=jax.ShapeDtypeStruct((N, D_OUT), jnp.float32),
)


def kernel(features, edge_index, W1, b1, W2, b2):
    src = edge_index[0].reshape(NW, K, B)
    dst = edge_index[1].reshape(NW, K, B)

    deg_parts = _sc_degrees(src, dst)
    scales = _tc_scales(deg_parts)
    s_out = scales[0, :N].reshape(N, 1)
    s_in = scales[1, :N].reshape(N, 1)

    t1 = _tc_layer1(features, s_out, W1)
    p1 = _sc_aggregate(t1, src, dst)
    t2 = _tc_mid(p1[0, :N], p1[1, :N], s_in, s_out, b1.reshape(1, HID))
    p2 = _sc_aggregate(t2, src, dst)
    return _tc_layer2(p2[0, :N], p2[1, :N], W2, s_in, b2.reshape(1, D_OUT))


# speedup vs baseline: 14.6184x; 1.4247x over previous
import functools

import jax
import jax.numpy as jnp
from jax import lax
from jax.experimental import pallas as pl
from jax.experimental.pallas import tpu as pltpu
from jax.experimental.pallas import tpu_sc as plsc

N = 10000
E = 320000
D_IN = 128
HID = 16
D_OUT = 128

NC = 2
NS = 16
NW = NC * NS
EPT = E // NW
B = 80
K = EPT // B
BA = 100
KA = EPT // BA
PAD_N = 10240
RPT = PAD_N // NS

_mesh = plsc.VectorSubcoreMesh(
    core_axis_name="c", subcore_axis_name="s", num_cores=NC, num_subcores=NS
)
_sc_params = pltpu.CompilerParams(use_tc_tiling_on_sc=False)


@functools.partial(
    pl.kernel,
    out_type=jax.ShapeDtypeStruct((NC, 2, PAD_N), jnp.float32),
    mesh=_mesh,
    compiler_params=_sc_params,
    scratch_types=[
        pltpu.VMEM((K, B), jnp.int32),
        pltpu.VMEM((K, B), jnp.int32),
        pltpu.VMEM((B,), jnp.float32),
        pltpu.VMEM((RPT,), jnp.float32),
        pltpu.VMEM_SHARED((PAD_N,), jnp.float32),
        pltpu.VMEM_SHARED((PAD_N,), jnp.float32),
    ],
)
def _sc_degrees(src_hbm, dst_hbm, out_hbm, sidx, didx, ones_v, zbuf,
                dout_sh, din_sh):
    cid = lax.axis_index("c")
    sid = lax.axis_index("s")
    wid = cid * NS + sid

    zero16 = jnp.zeros((16,), jnp.float32)
    one16 = jnp.ones((16,), jnp.float32)

    def fill_z(i, _):
        zbuf[pl.ds(i * 16, 16)] = zero16
        return 0
    lax.fori_loop(0, RPT // 16, fill_z, 0)
    for i in range(B // 16):
        ones_v[pl.ds(i * 16, 16)] = one16

    pltpu.sync_copy(zbuf, dout_sh.at[pl.ds(sid * RPT, RPT)])
    pltpu.sync_copy(zbuf, din_sh.at[pl.ds(sid * RPT, RPT)])
    plsc.subcore_barrier()

    pltpu.sync_copy(src_hbm.at[wid], sidx)
    pltpu.sync_copy(dst_hbm.at[wid], didx)

    def step(j, _):
        pltpu.sync_copy(ones_v, dout_sh.at[sidx.at[j]], add=True)
        pltpu.sync_copy(ones_v, din_sh.at[didx.at[j]], add=True)
        return 0
    lax.fori_loop(0, K, step, 0)
    plsc.subcore_barrier()

    pltpu.sync_copy(dout_sh.at[pl.ds(sid * RPT, RPT)], zbuf)
    pltpu.sync_copy(zbuf, out_hbm.at[cid, 0, pl.ds(sid * RPT, RPT)])
    pltpu.sync_copy(din_sh.at[pl.ds(sid * RPT, RPT)], zbuf)
    pltpu.sync_copy(zbuf, out_hbm.at[cid, 1, pl.ds(sid * RPT, RPT)])


@functools.partial(
    pl.kernel,
    out_type=jax.ShapeDtypeStruct((NC, PAD_N, HID), jnp.float32),
    mesh=_mesh,
    compiler_params=_sc_params,
    scratch_types=[
        pltpu.VMEM((KA, BA), jnp.int32),
        pltpu.VMEM((KA, BA), jnp.int32),
        pltpu.VMEM((BA, HID), jnp.float32),
        pltpu.VMEM((BA, HID), jnp.float32),
        pltpu.VMEM((RPT, HID), jnp.float32),
        pltpu.VMEM_SHARED((PAD_N, HID), jnp.float32),
        pltpu.SemaphoreType.DMA,
        pltpu.SemaphoreType.DMA,
    ],
)
def _sc_aggregate(table_hbm, src_hbm, dst_hbm, out_hbm, sidx, didx, msg0,
                  msg1, zbuf, agg_sh, sem0, sem1):
    cid = lax.axis_index("c")
    sid = lax.axis_index("s")
    wid = cid * NS + sid

    zero16 = jnp.zeros((16,), jnp.float32)

    def fill_z(i, _):
        zbuf[i, :] = zero16
        return 0
    lax.fori_loop(0, RPT, fill_z, 0)

    pltpu.sync_copy(zbuf, agg_sh.at[pl.ds(sid * RPT, RPT)])
    plsc.subcore_barrier()

    pltpu.sync_copy(src_hbm.at[wid], sidx)
    pltpu.sync_copy(dst_hbm.at[wid], didx)

    pltpu.async_copy(table_hbm.at[sidx.at[0]], msg0, sem0)

    def step(i, _):
        j0 = 2 * i
        j1 = j0 + 1
        cp1 = pltpu.async_copy(table_hbm.at[sidx.at[j1]], msg1, sem1)
        pltpu.make_async_copy(table_hbm.at[sidx.at[j0]], msg0, sem0).wait()
        pltpu.sync_copy(msg0, agg_sh.at[didx.at[j0]], add=True)

        @pl.when(j1 + 1 < KA)
        def _():
            pltpu.async_copy(table_hbm.at[sidx.at[j1 + 1]], msg0, sem0)

        cp1.wait()
        pltpu.sync_copy(msg1, agg_sh.at[didx.at[j1]], add=True)
        return 0
    lax.fori_loop(0, KA // 2, step, 0)
    plsc.subcore_barrier()

    pltpu.sync_copy(agg_sh.at[pl.ds(sid * RPT, RPT)], zbuf)
    pltpu.sync_copy(zbuf, out_hbm.at[cid, pl.ds(sid * RPT, RPT)])


def _scale(pa, pb):
    return lax.rsqrt(jnp.maximum(pa + pb, 1.0))


def _tc_layer1_body(x_ref, po0_ref, po1_ref, w_ref, o_ref):
    x = x_ref[...] * _scale(po0_ref[...], po1_ref[...])
    o_ref[...] = jnp.dot(x, w_ref[...], preferred_element_type=jnp.float32)


_tc_layer1 = pl.pallas_call(
    _tc_layer1_body,
    out_shape=jax.ShapeDtypeStruct((N, HID), jnp.float32),
)


def _tc_mid_body(a_ref, b_ref, pi0_ref, pi1_ref, po0_ref, po1_ref, bias_ref,
                 o_ref):
    s_in = _scale(pi0_ref[...], pi1_ref[...])
    s_out = _scale(po0_ref[...], po1_ref[...])
    h = (a_ref[...] + b_ref[...]) * s_in + bias_ref[...]
    o_ref[...] = jnp.maximum(h, 0.0) * s_out


_tc_mid = pl.pallas_call(
    _tc_mid_body,
    out_shape=jax.ShapeDtypeStruct((N, HID), jnp.float32),
)


def _tc_layer2_body(a_ref, b_ref, w_ref, pi0_ref, pi1_ref, bias_ref, o_ref):
    agg = a_ref[...] + b_ref[...]
    y = jnp.dot(agg, w_ref[...], preferred_element_type=jnp.float32)
    o_ref[...] = y * _scale(pi0_ref[...], pi1_ref[...]) + bias_ref[...]


_tc_layer2 = pl.pallas_call(
    _tc_layer2_body,
    out_shape=jax.ShapeDtypeStruct((N, D_OUT), jnp.float32),
)


def kernel(features, edge_index, W1, b1, W2, b2):
    src_d = edge_index[0].reshape(NW, K, B)
    dst_d = edge_index[1].reshape(NW, K, B)
    src_a = edge_index[0].reshape(NW, KA, BA)
    dst_a = edge_index[1].reshape(NW, KA, BA)

    deg = _sc_degrees(src_d, dst_d)
    po0 = deg[0, 0, :N].reshape(N, 1)
    po1 = deg[1, 0, :N].reshape(N, 1)
    pi0 = deg[0, 1, :N].reshape(N, 1)
    pi1 = deg[1, 1, :N].reshape(N, 1)

    t1 = _tc_layer1(features, po0, po1, W1)
    p1 = _sc_aggregate(t1, src_a, dst_a)
    t2 = _tc_mid(p1[0, :N], p1[1, :N], pi0, pi1, po0, po1,
                 b1.reshape(1, HID))
    p2 = _sc_aggregate(t2, src_a, dst_a)
    return _tc_layer2(p2[0, :N], p2[1, :N], W2, pi0, pi1,
                      b2.reshape(1, D_OUT))


# speedup vs baseline: 16.1270x; 1.1032x over previous
import functools

import jax
import jax.numpy as jnp
from jax import lax
from jax.experimental import pallas as pl
from jax.experimental.pallas import tpu as pltpu
from jax.experimental.pallas import tpu_sc as plsc

N = 10000
E = 320000
D_IN = 128
HID = 16
D_OUT = 128

NC = 2
NS = 16
NW = NC * NS
EPT = E // NW
B = 100
K = EPT // B
DW = 8
PAD_N = 10240
RPT = PAD_N // NS

_mesh = plsc.VectorSubcoreMesh(
    core_axis_name="c", subcore_axis_name="s", num_cores=NC, num_subcores=NS
)
_sc_params = pltpu.CompilerParams(use_tc_tiling_on_sc=False)


@functools.partial(
    pl.kernel,
    out_type=jax.ShapeDtypeStruct((NC, 2, PAD_N), jnp.float32),
    mesh=_mesh,
    compiler_params=_sc_params,
    scratch_types=[
        pltpu.VMEM((K, B), jnp.int32),
        pltpu.VMEM((K, B), jnp.int32),
        pltpu.VMEM((112,), jnp.float32),
        pltpu.VMEM((RPT,), jnp.float32),
        pltpu.VMEM_SHARED((PAD_N,), jnp.float32),
        pltpu.VMEM_SHARED((PAD_N,), jnp.float32),
        pltpu.SemaphoreType.DMA,
        pltpu.SemaphoreType.DMA,
    ],
)
def _sc_degrees(src_hbm, dst_hbm, out_hbm, sidx, didx, ones_v, zbuf,
                dout_sh, din_sh, semo, semi):
    cid = lax.axis_index("c")
    sid = lax.axis_index("s")
    wid = cid * NS + sid

    zero16 = jnp.zeros((16,), jnp.float32)
    one16 = jnp.ones((16,), jnp.float32)

    def fill_z(i, _):
        zbuf[pl.ds(i * 16, 16)] = zero16
        return 0
    lax.fori_loop(0, RPT // 16, fill_z, 0)
    for i in range(7):
        ones_v[pl.ds(i * 16, 16)] = one16

    pltpu.sync_copy(zbuf, dout_sh.at[pl.ds(sid * RPT, RPT)])
    pltpu.sync_copy(zbuf, din_sh.at[pl.ds(sid * RPT, RPT)])
    plsc.subcore_barrier()

    pltpu.sync_copy(src_hbm.at[wid], sidx)
    pltpu.sync_copy(dst_hbm.at[wid], didx)

    def step(j, _):
        pltpu.async_copy(ones_v.at[pl.ds(0, B)], dout_sh.at[sidx.at[j]],
                         semo, add=True)
        pltpu.async_copy(ones_v.at[pl.ds(0, B)], din_sh.at[didx.at[j]],
                         semi, add=True)

        @pl.when(j >= DW)
        def _():
            pltpu.make_async_copy(ones_v.at[pl.ds(0, B)],
                                  dout_sh.at[sidx.at[j - DW]], semo).wait()
            pltpu.make_async_copy(ones_v.at[pl.ds(0, B)],
                                  din_sh.at[didx.at[j - DW]], semi).wait()
        return 0
    lax.fori_loop(0, K, step, 0)
    for w in range(DW):
        pltpu.make_async_copy(ones_v.at[pl.ds(0, B)],
                              dout_sh.at[sidx.at[K - DW + w]], semo).wait()
        pltpu.make_async_copy(ones_v.at[pl.ds(0, B)],
                              din_sh.at[didx.at[K - DW + w]], semi).wait()
    plsc.subcore_barrier()

    pltpu.sync_copy(dout_sh.at[pl.ds(sid * RPT, RPT)], zbuf)
    pltpu.sync_copy(zbuf, out_hbm.at[cid, 0, pl.ds(sid * RPT, RPT)])
    pltpu.sync_copy(din_sh.at[pl.ds(sid * RPT, RPT)], zbuf)
    pltpu.sync_copy(zbuf, out_hbm.at[cid, 1, pl.ds(sid * RPT, RPT)])


@functools.partial(
    pl.kernel,
    out_type=jax.ShapeDtypeStruct((NC, PAD_N, HID), jnp.float32),
    mesh=_mesh,
    compiler_params=_sc_params,
    scratch_types=[
        pltpu.VMEM((K, B), jnp.int32),
        pltpu.VMEM((K, B), jnp.int32),
        pltpu.VMEM((B, HID), jnp.float32),
        pltpu.VMEM((B, HID), jnp.float32),
        pltpu.VMEM((B, HID), jnp.float32),
        pltpu.VMEM((B, HID), jnp.float32),
        pltpu.VMEM((RPT, HID), jnp.float32),
        pltpu.VMEM_SHARED((PAD_N, HID), jnp.float32),
        pltpu.SemaphoreType.DMA,
        pltpu.SemaphoreType.DMA,
        pltpu.SemaphoreType.DMA,
        pltpu.SemaphoreType.DMA,
        pltpu.SemaphoreType.DMA,
        pltpu.SemaphoreType.DMA,
        pltpu.SemaphoreType.DMA,
        pltpu.SemaphoreType.DMA,
    ],
)
def _sc_aggregate(table_hbm, src_hbm, dst_hbm, out_hbm, sidx, didx, m0, m1,
                  m2, m3, zbuf, agg_sh, g0, g1, g2, g3, s0, s1, s2, s3):
    cid = lax.axis_index("c")
    sid = lax.axis_index("s")
    wid = cid * NS + sid
    msgs = (m0, m1, m2, m3)
    gsem = (g0, g1, g2, g3)
    ssem = (s0, s1, s2, s3)

    zero16 = jnp.zeros((16,), jnp.float32)

    def fill_z(i, _):
        zbuf[i, :] = zero16
        return 0
    lax.fori_loop(0, RPT, fill_z, 0)

    pltpu.sync_copy(zbuf, agg_sh.at[pl.ds(sid * RPT, RPT)])
    plsc.subcore_barrier()

    pltpu.sync_copy(src_hbm.at[wid], sidx)
    pltpu.sync_copy(dst_hbm.at[wid], didx)

    pltpu.async_copy(table_hbm.at[sidx.at[0]], m0, g0)
    pltpu.async_copy(table_hbm.at[sidx.at[1]], m1, g1)

    def step(i, _):
        for b in range(4):
            j = 4 * i + b
            bn = (b + 2) % 4
            pltpu.make_async_copy(table_hbm.at[sidx.at[j]], msgs[b],
                                  gsem[b]).wait()
            pltpu.async_copy(msgs[b], agg_sh.at[didx.at[j]], ssem[b],
                             add=True)

            @pl.when(j >= 2)
            def _():
                pltpu.make_async_copy(msgs[bn], agg_sh.at[didx.at[j - 2]],
                                      ssem[bn]).wait()

            @pl.when(j + 2 < K)
            def _():
                pltpu.async_copy(table_hbm.at[sidx.at[j + 2]], msgs[bn],
                                 gsem[bn])
        return 0
    lax.fori_loop(0, K // 4, step, 0)
    pltpu.make_async_copy(m2, agg_sh.at[didx.at[K - 2]], s2).wait()
    pltpu.make_async_copy(m3, agg_sh.at[didx.at[K - 1]], s3).wait()
    plsc.subcore_barrier()

    pltpu.sync_copy(agg_sh.at[pl.ds(sid * RPT, RPT)], zbuf)
    pltpu.sync_copy(zbuf, out_hbm.at[cid, pl.ds(sid * RPT, RPT)])


def _scale(pa, pb):
    return lax.rsqrt(jnp.maximum(pa + pb, 1.0))


def _tc_layer1_body(x_ref, po0_ref, po1_ref, w_ref, o_ref):
    x = x_ref[...] * _scale(po0_ref[...], po1_ref[...])
    o_ref[...] = jnp.dot(x, w_ref[...], preferred_element_type=jnp.float32)


_tc_layer1 = pl.pallas_call(
    _tc_layer1_body,
    out_shape=jax.ShapeDtypeStruct((N, HID), jnp.float32),
)


def _tc_mid_body(a_ref, b_ref, pi0_ref, pi1_ref, po0_ref, po1_ref, bias_ref,
                 o_ref):
    s_in = _scale(pi0_ref[...], pi1_ref[...])
    s_out = _scale(po0_ref[...], po1_ref[...])
    h = (a_ref[...] + b_ref[...]) * s_in + bias_ref[...]
    o_ref[...] = jnp.maximum(h, 0.0) * s_out


_tc_mid = pl.pallas_call(
    _tc_mid_body,
    out_shape=jax.ShapeDtypeStruct((N, HID), jnp.float32),
)


def _tc_layer2_body(a_ref, b_ref, w_ref, pi0_ref, pi1_ref, bias_ref, o_ref):
    agg = a_ref[...] + b_ref[...]
    y = jnp.dot(agg, w_ref[...], preferred_element_type=jnp.float32)
    o_ref[...] = y * _scale(pi0_ref[...], pi1_ref[...]) + bias_ref[...]


_tc_layer2 = pl.pallas_call(
    _tc_layer2_body,
    out_shape=jax.ShapeDtypeStruct((N, D_OUT), jnp.float32),
)


def kernel(features, edge_index, W1, b1, W2, b2):
    src_a = edge_index[0].reshape(NW, K, B)
    dst_a = edge_index[1].reshape(NW, K, B)

    deg = _sc_degrees(src_a, dst_a)
    po0 = deg[0, 0, :N].reshape(N, 1)
    po1 = deg[1, 0, :N].reshape(N, 1)
    pi0 = deg[0, 1, :N].reshape(N, 1)
    pi1 = deg[1, 1, :N].reshape(N, 1)

    t1 = _tc_layer1(features, po0, po1, W1)
    p1 = _sc_aggregate(t1, src_a, dst_a)
    t2 = _tc_mid(p1[0, :N], p1[1, :N], pi0, pi1, po0, po1,
                 b1.reshape(1, HID))
    p2 = _sc_aggregate(t2, src_a, dst_a)
    return _tc_layer2(p2[0, :N], p2[1, :N], W2, pi0, pi1,
                      b2.reshape(1, D_OUT))


# speedup vs baseline: 20.2649x; 1.2566x over previous
import functools

import jax
import jax.numpy as jnp
from jax import lax
from jax.experimental import pallas as pl
from jax.experimental.pallas import tpu as pltpu
from jax.experimental.pallas import tpu_sc as plsc

N = 10000
E = 320000
D_IN = 128
HID = 16
D_OUT = 128

NC = 2
NS = 16
NW = NC * NS
EPT = E // NW
B = 100
K = EPT // B
DW = 8
PAD_N = 10240
RPT = PAD_N // NS
PR = PAD_N // HID
NPR = N // HID
PKW = HID * HID

_mesh = plsc.VectorSubcoreMesh(
    core_axis_name="c", subcore_axis_name="s", num_cores=NC, num_subcores=NS
)
_sc_params = pltpu.CompilerParams(use_tc_tiling_on_sc=False)


@functools.partial(
    pl.kernel,
    out_type=jax.ShapeDtypeStruct((NC, 2, PAD_N), jnp.float32),
    mesh=_mesh,
    compiler_params=_sc_params,
    scratch_types=[
        pltpu.VMEM((K, B), jnp.int32),
        pltpu.VMEM((K, B), jnp.int32),
        pltpu.VMEM((112,), jnp.float32),
        pltpu.VMEM((RPT,), jnp.float32),
        pltpu.VMEM_SHARED((PAD_N,), jnp.float32),
        pltpu.VMEM_SHARED((PAD_N,), jnp.float32),
        pltpu.SemaphoreType.DMA,
        pltpu.SemaphoreType.DMA,
    ],
)
def _sc_degrees(ei_hbm, out_hbm, sidx, didx, ones_v, zbuf,
                dout_sh, din_sh, semo, semi):
    cid = lax.axis_index("c")
    sid = lax.axis_index("s")
    wid = cid * NS + sid

    zero16 = jnp.zeros((16,), jnp.float32)
    one16 = jnp.ones((16,), jnp.float32)

    def fill_z(i, _):
        zbuf[pl.ds(i * 16, 16)] = zero16
        return 0
    lax.fori_loop(0, RPT // 16, fill_z, 0)
    for i in range(7):
        ones_v[pl.ds(i * 16, 16)] = one16

    pltpu.sync_copy(zbuf, dout_sh.at[pl.ds(sid * RPT, RPT)])
    pltpu.sync_copy(zbuf, din_sh.at[pl.ds(sid * RPT, RPT)])
    plsc.subcore_barrier()

    pltpu.sync_copy(ei_hbm.at[0, wid], sidx)
    pltpu.sync_copy(ei_hbm.at[1, wid], didx)

    def step(j, _):
        pltpu.async_copy(ones_v.at[pl.ds(0, B)], dout_sh.at[sidx.at[j]],
                         semo, add=True)
        pltpu.async_copy(ones_v.at[pl.ds(0, B)], din_sh.at[didx.at[j]],
                         semi, add=True)

        @pl.when(j >= DW)
        def _():
            pltpu.make_async_copy(ones_v.at[pl.ds(0, B)],
                                  dout_sh.at[sidx.at[j - DW]], semo).wait()
            pltpu.make_async_copy(ones_v.at[pl.ds(0, B)],
                                  din_sh.at[didx.at[j - DW]], semi).wait()
        return 0
    lax.fori_loop(0, K, step, 0)
    for w in range(DW):
        pltpu.make_async_copy(ones_v.at[pl.ds(0, B)],
                              dout_sh.at[sidx.at[K - DW + w]], semo).wait()
        pltpu.make_async_copy(ones_v.at[pl.ds(0, B)],
                              din_sh.at[didx.at[K - DW + w]], semi).wait()
    plsc.subcore_barrier()

    pltpu.sync_copy(dout_sh.at[pl.ds(sid * RPT, RPT)], zbuf)
    pltpu.sync_copy(zbuf, out_hbm.at[cid, 0, pl.ds(sid * RPT, RPT)])
    pltpu.sync_copy(din_sh.at[pl.ds(sid * RPT, RPT)], zbuf)
    pltpu.sync_copy(zbuf, out_hbm.at[cid, 1, pl.ds(sid * RPT, RPT)])


@functools.partial(
    pl.kernel,
    out_type=jax.ShapeDtypeStruct((NC, PAD_N, HID), jnp.float32),
    mesh=_mesh,
    compiler_params=_sc_params,
    scratch_types=[
        pltpu.VMEM((K, B), jnp.int32),
        pltpu.VMEM((K, B), jnp.int32),
        pltpu.VMEM((B, HID), jnp.float32),
        pltpu.VMEM((B, HID), jnp.float32),
        pltpu.VMEM((B, HID), jnp.float32),
        pltpu.VMEM((B, HID), jnp.float32),
        pltpu.VMEM((RPT, HID), jnp.float32),
        pltpu.VMEM_SHARED((PAD_N, HID), jnp.float32),
        pltpu.SemaphoreType.DMA,
        pltpu.SemaphoreType.DMA,
        pltpu.SemaphoreType.DMA,
        pltpu.SemaphoreType.DMA,
        pltpu.SemaphoreType.DMA,
        pltpu.SemaphoreType.DMA,
        pltpu.SemaphoreType.DMA,
        pltpu.SemaphoreType.DMA,
    ],
)
def _sc_aggregate(table_hbm, ei_hbm, out_hbm, sidx, didx, m0, m1,
                  m2, m3, zbuf, agg_sh, g0, g1, g2, g3, s0, s1, s2, s3):
    cid = lax.axis_index("c")
    sid = lax.axis_index("s")
    wid = cid * NS + sid
    msgs = (m0, m1, m2, m3)
    gsem = (g0, g1, g2, g3)
    ssem = (s0, s1, s2, s3)

    zero16 = jnp.zeros((16,), jnp.float32)

    def fill_z(i, _):
        zbuf[i, :] = zero16
        return 0
    lax.fori_loop(0, RPT, fill_z, 0)

    pltpu.sync_copy(zbuf, agg_sh.at[pl.ds(sid * RPT, RPT)])
    plsc.subcore_barrier()

    pltpu.sync_copy(ei_hbm.at[0, wid], sidx)
    pltpu.sync_copy(ei_hbm.at[1, wid], didx)

    pltpu.async_copy(table_hbm.at[sidx.at[0]], m0, g0)
    pltpu.async_copy(table_hbm.at[sidx.at[1]], m1, g1)

    def step(i, _):
        for b in range(4):
            j = 4 * i + b
            bn = (b + 2) % 4
            pltpu.make_async_copy(table_hbm.at[sidx.at[j]], msgs[b],
                                  gsem[b]).wait()
            pltpu.async_copy(msgs[b], agg_sh.at[didx.at[j]], ssem[b],
                             add=True)

            @pl.when(j >= 2)
            def _():
                pltpu.make_async_copy(msgs[bn], agg_sh.at[didx.at[j - 2]],
                                      ssem[bn]).wait()

            @pl.when(j + 2 < K)
            def _():
                pltpu.async_copy(table_hbm.at[sidx.at[j + 2]], msgs[bn],
                                 gsem[bn])
        return 0
    lax.fori_loop(0, K // 4, step, 0)
    pltpu.make_async_copy(m2, agg_sh.at[didx.at[K - 2]], s2).wait()
    pltpu.make_async_copy(m3, agg_sh.at[didx.at[K - 1]], s3).wait()
    plsc.subcore_barrier()

    pltpu.sync_copy(agg_sh.at[pl.ds(sid * RPT, RPT)], zbuf)
    pltpu.sync_copy(zbuf, out_hbm.at[cid, pl.ds(sid * RPT, RPT)])


def _scale(pa, pb):
    return lax.rsqrt(jnp.maximum(pa + pb, 1.0))


def _replicate(s16, width):
    rows = s16.shape[0]
    s3 = lax.broadcast_in_dim(s16, (rows, HID, width), (0, 1))
    return jnp.reshape(s3, (rows, HID * width))


def _tc_layer1_body(fp_ref, w_ref, dg_ref, o_ref):
    s16 = _scale(dg_ref[0, 0, :NPR, :], dg_ref[1, 0, :NPR, :])
    srep = _replicate(s16, D_IN)
    x = fp_ref[...] * srep
    o_ref[...] = jnp.dot(x, w_ref[...], preferred_element_type=jnp.float32)


_tc_layer1 = pl.pallas_call(
    _tc_layer1_body,
    out_shape=jax.ShapeDtypeStruct((NPR, PKW), jnp.float32),
)


def _tc_mid_body(p_ref, dg_ref, bias_ref, o_ref):
    s_in = _replicate(_scale(dg_ref[0, 1], dg_ref[1, 1]), HID)
    s_out = _replicate(_scale(dg_ref[0, 0], dg_ref[1, 0]), HID)
    h = (p_ref[0] + p_ref[1]) * s_in + bias_ref[...]
    o_ref[...] = jnp.maximum(h, 0.0) * s_out


_tc_mid = pl.pallas_call(
    _tc_mid_body,
    out_shape=jax.ShapeDtypeStruct((PR, PKW), jnp.float32),
)


def _tc_layer2_body(p_ref, w_ref, pi0_ref, pi1_ref, bias_ref, o_ref):
    agg = p_ref[0, :N] + p_ref[1, :N]
    y = jnp.dot(agg, w_ref[...], preferred_element_type=jnp.float32)
    o_ref[...] = y * _scale(pi0_ref[...], pi1_ref[...]) + bias_ref[...]


_tc_layer2 = pl.pallas_call(
    _tc_layer2_body,
    out_shape=jax.ShapeDtypeStruct((N, D_OUT), jnp.float32),
)


def kernel(features, edge_index, W1, b1, W2, b2):
    ei = edge_index.reshape(2, NW, K, B)

    deg = _sc_degrees(ei)
    dgp = deg.reshape(2, 2, PR, HID)
    pi0 = deg[0, 1, :N].reshape(N, 1)
    pi1 = deg[1, 1, :N].reshape(N, 1)

    fp = features.reshape(NPR, HID * D_IN)
    w1p = jnp.kron(jnp.eye(HID, dtype=W1.dtype), W1)
    t1p = _tc_layer1(fp, w1p, dgp)

    p1 = _sc_aggregate(t1p.reshape(N, HID), ei)
    t2p = _tc_mid(p1.reshape(2, PR, PKW), dgp,
                  jnp.tile(b1, HID).reshape(1, PKW))

    p2 = _sc_aggregate(t2p.reshape(PAD_N, HID), ei)
    return _tc_layer2(p2, W2, pi0, pi1, b2.reshape(1, D_OUT))


# speedup vs baseline: 27.1176x; 1.3382x over previous
import functools

import jax
import jax.numpy as jnp
from jax import lax
from jax.experimental import pallas as pl
from jax.experimental.pallas import tpu as pltpu
from jax.experimental.pallas import tpu_sc as plsc

N = 10000
E = 320000
D_IN = 128
HID = 16
D_OUT = 128

NC = 2
NS = 16
NW = NC * NS
EPT = E // NW
B = 125
K = EPT // B
DW = 8
PAD_N = 10240
RPT = PAD_N // NS
PR = PAD_N // HID
NPR = N // HID
PKW = HID * HID

_mesh = plsc.VectorSubcoreMesh(
    core_axis_name="c", subcore_axis_name="s", num_cores=NC, num_subcores=NS
)
_sc_params = pltpu.CompilerParams(use_tc_tiling_on_sc=False)


@functools.partial(
    pl.kernel,
    out_type=jax.ShapeDtypeStruct((NC, 2, PAD_N), jnp.float32),
    mesh=_mesh,
    compiler_params=_sc_params,
    scratch_types=[
        pltpu.VMEM((K, B), jnp.int32),
        pltpu.VMEM((K, B), jnp.int32),
        pltpu.VMEM((128,), jnp.float32),
        pltpu.VMEM((RPT,), jnp.float32),
        pltpu.VMEM_SHARED((PAD_N,), jnp.float32),
        pltpu.VMEM_SHARED((PAD_N,), jnp.float32),
        pltpu.SemaphoreType.DMA,
        pltpu.SemaphoreType.DMA,
    ],
)
def _sc_degrees(ei_hbm, out_hbm, sidx, didx, ones_v, zbuf,
                dout_sh, din_sh, semo, semi):
    cid = lax.axis_index("c")
    sid = lax.axis_index("s")
    wid = cid * NS + sid

    zero16 = jnp.zeros((16,), jnp.float32)
    one16 = jnp.ones((16,), jnp.float32)

    def fill_z(i, _):
        zbuf[pl.ds(i * 16, 16)] = zero16
        return 0
    lax.fori_loop(0, RPT // 16, fill_z, 0)
    for i in range(8):
        ones_v[pl.ds(i * 16, 16)] = one16

    pltpu.sync_copy(zbuf, dout_sh.at[pl.ds(sid * RPT, RPT)])
    pltpu.sync_copy(zbuf, din_sh.at[pl.ds(sid * RPT, RPT)])
    plsc.subcore_barrier()

    pltpu.sync_copy(ei_hbm.at[0, wid], sidx)
    pltpu.sync_copy(ei_hbm.at[1, wid], didx)

    def step(j, _):
        pltpu.async_copy(ones_v.at[pl.ds(0, B)], dout_sh.at[sidx.at[j]],
                         semo, add=True)
        pltpu.async_copy(ones_v.at[pl.ds(0, B)], din_sh.at[didx.at[j]],
                         semi, add=True)

        @pl.when(j >= DW)
        def _():
            pltpu.make_async_copy(ones_v.at[pl.ds(0, B)],
                                  dout_sh.at[sidx.at[j - DW]], semo).wait()
            pltpu.make_async_copy(ones_v.at[pl.ds(0, B)],
                                  din_sh.at[didx.at[j - DW]], semi).wait()
        return 0
    lax.fori_loop(0, K, step, 0)
    for w in range(DW):
        pltpu.make_async_copy(ones_v.at[pl.ds(0, B)],
                              dout_sh.at[sidx.at[K - DW + w]], semo).wait()
        pltpu.make_async_copy(ones_v.at[pl.ds(0, B)],
                              din_sh.at[didx.at[K - DW + w]], semi).wait()
    plsc.subcore_barrier()

    pltpu.sync_copy(dout_sh.at[pl.ds(sid * RPT, RPT)], zbuf)
    pltpu.sync_copy(zbuf, out_hbm.at[cid, 0, pl.ds(sid * RPT, RPT)])
    pltpu.sync_copy(din_sh.at[pl.ds(sid * RPT, RPT)], zbuf)
    pltpu.sync_copy(zbuf, out_hbm.at[cid, 1, pl.ds(sid * RPT, RPT)])


@functools.partial(
    pl.kernel,
    out_type=jax.ShapeDtypeStruct((NC, PAD_N, HID), jnp.float32),
    mesh=_mesh,
    compiler_params=_sc_params,
    scratch_types=[
        pltpu.VMEM((K, B), jnp.int32),
        pltpu.VMEM((K, B), jnp.int32),
        pltpu.VMEM((8, B, HID), jnp.float32),
        pltpu.VMEM((RPT, HID), jnp.float32),
        pltpu.VMEM_SHARED((PAD_N, HID), jnp.float32),
        pltpu.SemaphoreType.DMA,
        pltpu.SemaphoreType.DMA,
        pltpu.SemaphoreType.DMA,
        pltpu.SemaphoreType.DMA,
        pltpu.SemaphoreType.DMA,
        pltpu.SemaphoreType.DMA,
        pltpu.SemaphoreType.DMA,
        pltpu.SemaphoreType.DMA,
        pltpu.SemaphoreType.DMA,
        pltpu.SemaphoreType.DMA,
        pltpu.SemaphoreType.DMA,
        pltpu.SemaphoreType.DMA,
        pltpu.SemaphoreType.DMA,
        pltpu.SemaphoreType.DMA,
        pltpu.SemaphoreType.DMA,
        pltpu.SemaphoreType.DMA,
    ],
)
def _sc_aggregate(table_hbm, ei_hbm, out_hbm, sidx, didx, mring, zbuf, agg_sh,
                  g0, g1, g2, g3, g4, g5, g6, g7,
                  s0, s1, s2, s3, s4, s5, s6, s7):
    cid = lax.axis_index("c")
    sid = lax.axis_index("s")
    wid = cid * NS + sid
    msgs = tuple(mring.at[b] for b in range(8))
    gsem = (g0, g1, g2, g3, g4, g5, g6, g7)
    ssem = (s0, s1, s2, s3, s4, s5, s6, s7)

    zero16 = jnp.zeros((16,), jnp.float32)

    def fill_z(i, _):
        zbuf[i, :] = zero16
        return 0
    lax.fori_loop(0, RPT, fill_z, 0)

    pltpu.sync_copy(zbuf, agg_sh.at[pl.ds(sid * RPT, RPT)])
    plsc.subcore_barrier()

    pltpu.sync_copy(ei_hbm.at[0, wid], sidx)
    pltpu.sync_copy(ei_hbm.at[1, wid], didx)

    for j0 in range(4):
        pltpu.async_copy(table_hbm.at[sidx.at[j0]], msgs[j0], gsem[j0])

    def step(i, _):
        for b in range(8):
            j = 8 * i + b
            bn = (b + 4) % 8
            pltpu.make_async_copy(table_hbm.at[sidx.at[j]], msgs[b],
                                  gsem[b]).wait()
            pltpu.async_copy(msgs[b], agg_sh.at[didx.at[j]], ssem[b],
                             add=True)

            @pl.when(j >= 4)
            def _():
                pltpu.make_async_copy(msgs[bn], agg_sh.at[didx.at[j - 4]],
                                      ssem[bn]).wait()

            @pl.when(j + 4 < K)
            def _():
                pltpu.async_copy(table_hbm.at[sidx.at[j + 4]], msgs[bn],
                                 gsem[bn])
        return 0
    lax.fori_loop(0, K // 8, step, 0)
    for j0 in range(K - 4, K):
        b = j0 % 8
        pltpu.make_async_copy(msgs[b], agg_sh.at[didx.at[j0]],
                              ssem[b]).wait()
    plsc.subcore_barrier()

    pltpu.sync_copy(agg_sh.at[pl.ds(sid * RPT, RPT)], zbuf)
    pltpu.sync_copy(zbuf, out_hbm.at[cid, pl.ds(sid * RPT, RPT)])


def _scale(pa, pb):
    return lax.rsqrt(jnp.maximum(pa + pb, 1.0))


def _replicate(s16, width):
    rows = s16.shape[0]
    s3 = lax.broadcast_in_dim(s16, (rows, HID, width), (0, 1))
    return jnp.reshape(s3, (rows, HID * width))


def _tc_layer1_body(fp_ref, w_ref, dg_ref, o_ref):
    s16 = _scale(dg_ref[0, 0, :NPR, :], dg_ref[1, 0, :NPR, :])
    srep = _replicate(s16, D_IN)
    x = fp_ref[...] * srep
    o_ref[...] = jnp.dot(x, w_ref[...], preferred_element_type=jnp.float32)


_tc_layer1 = pl.pallas_call(
    _tc_layer1_body,
    out_shape=jax.ShapeDtypeStruct((NPR, PKW), jnp.float32),
)


def _tc_mid_body(p_ref, dg_ref, bias_ref, o_ref):
    s_in = _replicate(_scale(dg_ref[0, 1], dg_ref[1, 1]), HID)
    s_out = _replicate(_scale(dg_ref[0, 0], dg_ref[1, 0]), HID)
    h = (p_ref[0] + p_ref[1]) * s_in + bias_ref[...]
    o_ref[...] = jnp.maximum(h, 0.0) * s_out


_tc_mid = pl.pallas_call(
    _tc_mid_body,
    out_shape=jax.ShapeDtypeStruct((PR, PKW), jnp.float32),
)


def _tc_layer2_body(p_ref, w_ref, dg_ref, bias_ref, o_ref):
    agg = p_ref[0] + p_ref[1]
    y = jnp.dot(agg, w_ref[...], preferred_element_type=jnp.float32)
    s_in = _replicate(_scale(dg_ref[0, 1], dg_ref[1, 1]), D_OUT)
    o_ref[...] = y * s_in + bias_ref[...]


_tc_layer2 = pl.pallas_call(
    _tc_layer2_body,
    out_shape=jax.ShapeDtypeStruct((PR, HID * D_OUT), jnp.float32),
)


def kernel(features, edge_index, W1, b1, W2, b2):
    ei = edge_index.reshape(2, NW, K, B)

    deg = _sc_degrees(ei)
    dgp = deg.reshape(2, 2, PR, HID)

    fp = features.reshape(NPR, HID * D_IN)
    w1p = jnp.kron(jnp.eye(HID, dtype=W1.dtype), W1)
    t1p = _tc_layer1(fp, w1p, dgp)

    p1 = _sc_aggregate(t1p.reshape(N, HID), ei)
    t2p = _tc_mid(p1.reshape(2, PR, PKW), dgp,
                  jnp.tile(b1, HID).reshape(1, PKW))

    p2 = _sc_aggregate(t2p.reshape(PAD_N, HID), ei)
    w2p = jnp.kron(jnp.eye(HID, dtype=W2.dtype), W2)
    outp = _tc_layer2(p2.reshape(2, PR, PKW), w2p, dgp,
                      jnp.tile(b2, HID).reshape(1, HID * D_OUT))
    return outp.reshape(PAD_N, D_OUT)[:N]
